# baseline (speedup 1.0000x reference)
import functools

import jax
import jax.numpy as jnp
from jax import lax
from jax.experimental import pallas as pl
from jax.experimental.pallas import tpu as pltpu
from jax.experimental.pallas import tpu_sc as plsc

N_ROWS = 16906
DIM = 512

_info = plsc.get_sparse_core_info()
_NC, _NS = _info.num_cores, _info.num_subcores
_HALF0 = 8448
_HALF1 = N_ROWS - _HALF0
_CHUNK = 528
_REM = _HALF1 - _CHUNK * _NS
_CROWS = 48
_NCH = _CHUNK // _CROWS
_NBUF = 3

_mesh = plsc.VectorSubcoreMesh(core_axis_name="c", subcore_axis_name="s")


def _pipeline(nch, src_base, dst_base, table_hbm, out_ref, buf, in_sems, out_sems):
    in_d = [None] * nch
    out_d = [None] * nch

    def start_in(i):
        in_d[i] = pltpu.async_copy(
            table_hbm.at[pl.ds(src_base + i * _CROWS, _CROWS)],
            buf.at[i % _NBUF], in_sems[i % _NBUF])

    def start_out(i):
        out_d[i] = pltpu.async_copy(
            buf.at[i % _NBUF],
            out_ref.at[pl.ds(dst_base + i * _CROWS, _CROWS)],
            out_sems[i % _NBUF])

    for j in range(min(_NBUF - 1, nch)):
        start_in(j)
    for i in range(nch):
        j = i + _NBUF - 1
        if j < nch:
            if i >= 1:
                out_d[i - 1].wait()
            start_in(j)
        in_d[i].wait()
        start_out(i)
    for i in range(max(0, nch - _NBUF), nch):
        out_d[i].wait()


@functools.partial(
    pl.kernel,
    mesh=_mesh,
    out_type=(jax.ShapeDtypeStruct((_HALF0, DIM), jnp.float32),
              jax.ShapeDtypeStruct((_HALF1, DIM), jnp.float32)),
    scratch_types=[
        pltpu.VMEM((_NBUF, _CROWS, DIM), jnp.float32),
        pltpu.SemaphoreType.DMA,
        pltpu.SemaphoreType.DMA,
        pltpu.SemaphoreType.DMA,
        pltpu.SemaphoreType.DMA,
        pltpu.SemaphoreType.DMA,
        pltpu.SemaphoreType.DMA,
    ],
)
def _slice_copy(table_hbm, out0, out1, buf, si0, si1, si2, so0, so1, so2):
    cid = lax.axis_index("c")
    sid = lax.axis_index("s")
    in_sems = (si0, si1, si2)
    out_sems = (so0, so1, so2)

    @pl.when(cid == 0)
    def _core0():
        _pipeline(_NCH, sid * _CHUNK, sid * _CHUNK,
                  table_hbm, out0, buf, in_sems, out_sems)

    @pl.when(cid == 1)
    def _core1():
        _pipeline(_NCH, _HALF0 + sid * _CHUNK, sid * _CHUNK,
                  table_hbm, out1, buf, in_sems, out_sems)

        @pl.when(sid < _REM)
        def _tail():
            src = _HALF0 + _NS * _CHUNK + sid
            dst = _NS * _CHUNK + sid
            row = buf.at[0, pl.ds(0, 1)]
            pltpu.sync_copy(table_hbm.at[pl.ds(src, 1)], row)
            pltpu.sync_copy(row, out1.at[pl.ds(dst, 1)])


def kernel(x, table):
    del x
    o0, o1 = _slice_copy(table)
    return jnp.concatenate([o0, o1], axis=0)


# speedup vs baseline: 1.4932x; 1.4932x over previous
import functools

import jax
import jax.numpy as jnp
from jax import lax
from jax.experimental import pallas as pl
from jax.experimental.pallas import tpu as pltpu
from jax.experimental.pallas import tpu_sc as plsc

N_ROWS = 16906
DIM = 512

_info = plsc.get_sparse_core_info()
_NC, _NS = _info.num_cores, _info.num_subcores
_NW = _NC * _NS
_CHUNK = N_ROWS // _NW
_REM = N_ROWS - _CHUNK * _NW
_CROWS = 48
_NCH = _CHUNK // _CROWS
_NBUF = 4

_mesh = plsc.VectorSubcoreMesh(core_axis_name="c", subcore_axis_name="s")


@functools.partial(
    pl.kernel,
    mesh=_mesh,
    out_type=jax.ShapeDtypeStruct((N_ROWS, DIM), jnp.float32),
    scratch_types=[
        pltpu.VMEM((_NBUF, _CROWS, DIM), jnp.float32),
        pltpu.SemaphoreType.DMA,
        pltpu.SemaphoreType.DMA,
        pltpu.SemaphoreType.DMA,
        pltpu.SemaphoreType.DMA,
        pltpu.SemaphoreType.DMA,
        pltpu.SemaphoreType.DMA,
        pltpu.SemaphoreType.DMA,
        pltpu.SemaphoreType.DMA,
    ],
)
def _slice_copy(table_hbm, out_hbm, buf, si0, si1, si2, si3, so0, so1, so2, so3):
    wid = lax.axis_index("s") * _NC + lax.axis_index("c")
    base = wid * _CHUNK
    in_sems = (si0, si1, si2, si3)
    out_sems = (so0, so1, so2, so3)
    in_d = [None] * _NCH
    out_d = [None] * _NCH

    def start_in(i):
        off = base + i * _CROWS
        in_d[i] = pltpu.async_copy(
            table_hbm.at[pl.ds(off, _CROWS)], buf.at[i % _NBUF], in_sems[i % _NBUF])

    def start_out(i):
        off = base + i * _CROWS
        out_d[i] = pltpu.async_copy(
            buf.at[i % _NBUF], out_hbm.at[pl.ds(off, _CROWS)], out_sems[i % _NBUF])

    for j in range(_NBUF - 1):
        start_in(j)
    for i in range(_NCH):
        j = i + _NBUF - 1
        if j < _NCH:
            if i >= 1:
                out_d[i - 1].wait()
            start_in(j)
        in_d[i].wait()
        start_out(i)
    for i in range(max(0, _NCH - _NBUF), _NCH):
        out_d[i].wait()

    @pl.when(wid < _REM)
    def _tail():
        r = _NW * _CHUNK + wid
        row = buf.at[0, pl.ds(0, 1)]
        pltpu.sync_copy(table_hbm.at[pl.ds(r, 1)], row)
        pltpu.sync_copy(row, out_hbm.at[pl.ds(r, 1)])


def kernel(x, table):
    del x
    return _slice_copy(table)
